# baseline (speedup 1.0000x reference)
import functools

import jax
import jax.numpy as jnp
from jax.experimental import pallas as pl
from jax.experimental.pallas import tpu as pltpu

_Q_LEVELS = 32704
_COARSE_ITERS = 15
_FINE_ITERS = 5
_NACC = 8


def _fold8(a):
    a = a[:4] + a[4:8]
    a = a[:2] + a[2:4]
    return a[:1] + a[1:2]


def _count_ge_f32(pt, mid):
    rows, r = pt.shape
    mid8 = jnp.broadcast_to(mid, (8, r))
    accs = [jnp.zeros((8, r), jnp.float32) for _ in range(_NACC)]
    for c in range(rows // 8):
        sl = pt[8 * c:8 * c + 8]
        accs[c % _NACC] += (sl >= mid8).astype(jnp.float32)
    a = accs[0]
    for c in range(1, _NACC):
        a += accs[c]
    return _fold8(a)


def _count_ge_i16(q, mid_i):
    rows, r = q.shape
    mid16 = jnp.broadcast_to(mid_i.astype(jnp.int16), (16, r))
    one = jnp.int16(1)
    zero = jnp.int16(0)
    accs = [jnp.zeros((16, r), jnp.int16) for _ in range(_NACC)]
    for c in range(rows // 16):
        sl = q[16 * c:16 * c + 16]
        accs[c % _NACC] += jnp.where(sl >= mid16, one, zero)
    a = accs[0]
    for c in range(1, _NACC):
        a += accs[c]
    a = a[:8] + a[8:16]
    a = a[:4] + a[4:8]
    a = a[:2] + a[2:4]
    af = a.astype(jnp.float32)
    return af[:1] + af[1:2]


def _col_max(pt):
    rows, r = pt.shape
    accs = [jnp.zeros((8, r), jnp.float32) for _ in range(_NACC)]
    for c in range(rows // 8):
        sl = pt[8 * c:8 * c + 8]
        accs[c % _NACC] = jnp.maximum(accs[c % _NACC], sl)
    a = accs[0]
    for c in range(1, _NACC):
        a = jnp.maximum(a, accs[c])
    a = jnp.maximum(a[:4], a[4:8])
    a = jnp.maximum(a[:2], a[2:4])
    return jnp.maximum(a[:1], a[1:2])


def _sq_sum(enc):
    rows, r = enc.shape
    accs = [jnp.zeros((8, r), jnp.float32) for _ in range(_NACC)]
    for c in range(rows // 8):
        sl = enc[8 * c:8 * c + 8]
        accs[c % _NACC] += sl * sl
    a = accs[0]
    for c in range(1, _NACC):
        a += accs[c]
    return _fold8(a)


def _fused_kernel(min_ref, x_ref, w_ref, o_ref, *, k):
    xb = x_ref[...]
    xb = jnp.where(min_ref[0, 0] >= 0.0, xb * 2.0 - 1.0, xb)
    pt = jax.lax.dot_general(
        w_ref[...], xb,
        dimension_numbers=(((1,), (1,)), ((), ())),
        preferred_element_type=jnp.float32,
    )

    kf = jnp.float32(k)
    hi_g = jnp.maximum(_col_max(pt), 1e-30)
    scale = jnp.float32(_Q_LEVELS) / hi_g
    q = jnp.clip(pt * scale, 0.0, jnp.float32(_Q_LEVELS)).astype(
        jnp.int32).astype(jnp.int16)

    lo_i = jnp.zeros(hi_g.shape, jnp.int32)
    hi_i = jnp.full(hi_g.shape, 32768, jnp.int32)

    def coarse(_, carry):
        lo_i, hi_i = carry
        mid = (lo_i + hi_i) >> 1
        ge = _count_ge_i16(q, mid) >= kf
        return jnp.where(ge, mid, lo_i), jnp.where(ge, hi_i, mid)

    lo_i, _ = jax.lax.fori_loop(0, _COARSE_ITERS, coarse, (lo_i, hi_i))

    inv = hi_g * jnp.float32(1.0 / _Q_LEVELS)
    lo = lo_i.astype(jnp.float32) * inv * jnp.float32(1.0 - 1e-5)
    hi = (lo_i + 1).astype(jnp.float32) * inv * jnp.float32(1.0 + 1e-5) + 1e-30

    s2 = jnp.float32(_Q_LEVELS) / jnp.maximum(hi - lo, 1e-30)
    q2 = jnp.clip((pt - lo) * s2, 0.0, jnp.float32(_Q_LEVELS)).astype(
        jnp.int32).astype(jnp.int16)

    lo2 = jnp.zeros(hi_g.shape, jnp.int32)
    hi2 = jnp.full(hi_g.shape, 32768, jnp.int32)

    def fine(_, carry):
        lo2, hi2 = carry
        mid = (lo2 + hi2) >> 1
        ge = _count_ge_i16(q2, mid) >= kf
        return jnp.where(ge, mid, lo2), jnp.where(ge, hi2, mid)

    lo2, _ = jax.lax.fori_loop(0, _FINE_ITERS, fine, (lo2, hi2))

    inv2 = jnp.maximum(hi - lo, 1e-30) * jnp.float32(1.0 / _Q_LEVELS)
    thr = jnp.maximum(
        lo + lo2.astype(jnp.float32) * inv2 - (hi_g * 1e-6 + 1e-30), 0.0)

    enc = jnp.where(pt >= thr, pt, 0.0)
    nrm = jnp.sqrt(_sq_sum(enc))
    out_t = enc * (1.0 / jnp.maximum(nrm, 1e-12))
    o_ref[...] = out_t.T


def kernel(x, projection_weights):
    n, d_in = x.shape
    d_out = projection_weights.shape[0]
    k = max(1, int(d_out * 0.1))

    min_val = jnp.min(x).reshape(1, 1)

    block_rows = 512
    grid = (n // block_rows,)

    return pl.pallas_call(
        functools.partial(_fused_kernel, k=k),
        grid=grid,
        in_specs=[
            pl.BlockSpec(memory_space=pltpu.SMEM),
            pl.BlockSpec((block_rows, d_in), lambda i: (i, 0)),
            pl.BlockSpec((d_out, d_in), lambda i: (0, 0)),
        ],
        out_specs=pl.BlockSpec((block_rows, d_out), lambda i: (i, 0)),
        out_shape=jax.ShapeDtypeStruct((n, d_out), jnp.float32),
        compiler_params=pltpu.CompilerParams(
            dimension_semantics=("arbitrary",),
        ),
    )(min_val, x, projection_weights)


# speedup vs baseline: 1.0890x; 1.0890x over previous
import functools

import jax
import jax.numpy as jnp
from jax.experimental import pallas as pl
from jax.experimental.pallas import tpu as pltpu

_Q_LEVELS = 32704
_COARSE_ITERS = 15
_FINE_ITERS = 4
_NACC = 8


def _fold8(a):
    a = a[:4] + a[4:8]
    a = a[:2] + a[2:4]
    return a[:1] + a[1:2]


def _count_ge_f32(pt, mid):
    rows, r = pt.shape
    mid8 = jnp.broadcast_to(mid, (8, r))
    accs = [jnp.zeros((8, r), jnp.float32) for _ in range(_NACC)]
    for c in range(rows // 8):
        sl = pt[8 * c:8 * c + 8]
        accs[c % _NACC] += (sl >= mid8).astype(jnp.float32)
    a = accs[0]
    for c in range(1, _NACC):
        a += accs[c]
    return _fold8(a)


def _count_ge_i16(q, mid_i):
    rows, r = q.shape
    mid16 = jnp.broadcast_to(mid_i.astype(jnp.int16), (16, r))
    one = jnp.int16(1)
    zero = jnp.int16(0)
    accs = [jnp.zeros((16, r), jnp.int16) for _ in range(_NACC)]
    for c in range(rows // 16):
        sl = q[16 * c:16 * c + 16]
        accs[c % _NACC] += jnp.where(sl >= mid16, one, zero)
    a = accs[0]
    for c in range(1, _NACC):
        a += accs[c]
    a = a[:8] + a[8:16]
    a = a[:4] + a[4:8]
    a = a[:2] + a[2:4]
    af = a.astype(jnp.float32)
    return af[:1] + af[1:2]


def _col_max(pt):
    rows, r = pt.shape
    accs = [jnp.zeros((8, r), jnp.float32) for _ in range(_NACC)]
    for c in range(rows // 8):
        sl = pt[8 * c:8 * c + 8]
        accs[c % _NACC] = jnp.maximum(accs[c % _NACC], sl)
    a = accs[0]
    for c in range(1, _NACC):
        a = jnp.maximum(a, accs[c])
    a = jnp.maximum(a[:4], a[4:8])
    a = jnp.maximum(a[:2], a[2:4])
    return jnp.maximum(a[:1], a[1:2])


def _sq_sum(enc):
    rows, r = enc.shape
    accs = [jnp.zeros((8, r), jnp.float32) for _ in range(_NACC)]
    for c in range(rows // 8):
        sl = enc[8 * c:8 * c + 8]
        accs[c % _NACC] += sl * sl
    a = accs[0]
    for c in range(1, _NACC):
        a += accs[c]
    return _fold8(a)


def _fused_kernel(min_ref, x_ref, w_ref, o_ref, *, k):
    xb = x_ref[...]
    xb = jnp.where(min_ref[0, 0] >= 0.0, xb * 2.0 - 1.0, xb)
    pt = jax.lax.dot_general(
        w_ref[...], xb,
        dimension_numbers=(((1,), (1,)), ((), ())),
        preferred_element_type=jnp.float32,
    )

    kf = jnp.float32(k)
    hi_g = jnp.maximum(_col_max(pt), 1e-30)
    scale = jnp.float32(_Q_LEVELS) / hi_g
    q = jnp.clip(pt * scale, 0.0, jnp.float32(_Q_LEVELS)).astype(
        jnp.int32).astype(jnp.int16)

    lo_i = jnp.zeros(hi_g.shape, jnp.int32)
    hi_i = jnp.full(hi_g.shape, 32768, jnp.int32)

    def coarse(_, carry):
        lo_i, hi_i = carry
        mid = (lo_i + hi_i) >> 1
        ge = _count_ge_i16(q, mid) >= kf
        return jnp.where(ge, mid, lo_i), jnp.where(ge, hi_i, mid)

    lo_i, _ = jax.lax.fori_loop(0, _COARSE_ITERS, coarse, (lo_i, hi_i))

    inv = hi_g * jnp.float32(1.0 / _Q_LEVELS)
    lo = lo_i.astype(jnp.float32) * inv * jnp.float32(1.0 - 1e-5)
    hi = (lo_i + 1).astype(jnp.float32) * inv * jnp.float32(1.0 + 1e-5) + 1e-30

    def fine(_, carry):
        lo, hi = carry
        mid = 0.5 * (lo + hi)
        ge = _count_ge_f32(pt, mid) >= kf
        return jnp.where(ge, mid, lo), jnp.where(ge, hi, mid)

    thr, _ = jax.lax.fori_loop(0, _FINE_ITERS, fine, (lo, hi))

    enc = jnp.where(pt >= thr, pt, 0.0)
    nrm = jnp.sqrt(_sq_sum(enc))
    out_t = enc * (1.0 / jnp.maximum(nrm, 1e-12))
    o_ref[...] = out_t.T


def kernel(x, projection_weights):
    n, d_in = x.shape
    d_out = projection_weights.shape[0]
    k = max(1, int(d_out * 0.1))

    min_val = jnp.min(x).reshape(1, 1)

    block_rows = 512
    grid = (n // block_rows,)

    return pl.pallas_call(
        functools.partial(_fused_kernel, k=k),
        grid=grid,
        in_specs=[
            pl.BlockSpec(memory_space=pltpu.SMEM),
            pl.BlockSpec((block_rows, d_in), lambda i: (i, 0)),
            pl.BlockSpec((d_out, d_in), lambda i: (0, 0)),
        ],
        out_specs=pl.BlockSpec((block_rows, d_out), lambda i: (i, 0)),
        out_shape=jax.ShapeDtypeStruct((n, d_out), jnp.float32),
        compiler_params=pltpu.CompilerParams(
            dimension_semantics=("arbitrary",),
        ),
    )(min_val, x, projection_weights)


# speedup vs baseline: 1.1010x; 1.0110x over previous
import functools

import jax
import jax.numpy as jnp
from jax.experimental import pallas as pl
from jax.experimental.pallas import tpu as pltpu

_Q_LEVELS = 32704
_COARSE_ITERS = 15
_FINE_ITERS = 4
_NACC = 8


def _fold8(a):
    a = a[:4] + a[4:8]
    a = a[:2] + a[2:4]
    return a[:1] + a[1:2]


def _count_ge_f32(pt, mid):
    rows, r = pt.shape
    mid8 = jnp.broadcast_to(mid, (8, r))
    accs = [jnp.zeros((8, r), jnp.float32) for _ in range(_NACC)]
    for c in range(rows // 8):
        sl = pt[8 * c:8 * c + 8]
        accs[c % _NACC] += (sl >= mid8).astype(jnp.float32)
    a = accs[0]
    for c in range(1, _NACC):
        a += accs[c]
    return _fold8(a)


def _count_ge_i16(q, mid_i):
    rows, r = q.shape
    mid16 = jnp.broadcast_to(mid_i.astype(jnp.int16), (16, r))
    one = jnp.int16(1)
    zero = jnp.int16(0)
    accs = [jnp.zeros((16, r), jnp.int16) for _ in range(_NACC)]
    for c in range(rows // 16):
        sl = q[16 * c:16 * c + 16]
        accs[c % _NACC] += jnp.where(sl >= mid16, one, zero)
    a = accs[0]
    for c in range(1, _NACC):
        a += accs[c]
    a = a[:8] + a[8:16]
    a = a[:4] + a[4:8]
    a = a[:2] + a[2:4]
    af = a.astype(jnp.float32)
    return af[:1] + af[1:2]


def _col_max(pt):
    rows, r = pt.shape
    accs = [jnp.zeros((8, r), jnp.float32) for _ in range(_NACC)]
    for c in range(rows // 8):
        sl = pt[8 * c:8 * c + 8]
        accs[c % _NACC] = jnp.maximum(accs[c % _NACC], sl)
    a = accs[0]
    for c in range(1, _NACC):
        a = jnp.maximum(a, accs[c])
    a = jnp.maximum(a[:4], a[4:8])
    a = jnp.maximum(a[:2], a[2:4])
    return jnp.maximum(a[:1], a[1:2])


def _sq_sum_masked(pt, thr):
    rows, r = pt.shape
    thr8 = jnp.broadcast_to(thr, (8, r))
    zero = jnp.float32(0.0)
    accs = [jnp.zeros((8, r), jnp.float32) for _ in range(_NACC)]
    for c in range(rows // 8):
        sl = pt[8 * c:8 * c + 8]
        e = jnp.where(sl >= thr8, sl, zero)
        accs[c % _NACC] += e * e
    a = accs[0]
    for c in range(1, _NACC):
        a += accs[c]
    return _fold8(a)


def _fused_kernel(min_ref, x_ref, w_ref, o_ref, *, k):
    xb = x_ref[...]
    xb = jnp.where(min_ref[0, 0] >= 0.0, xb * 2.0 - 1.0, xb)
    pt = jax.lax.dot_general(
        w_ref[...], xb,
        dimension_numbers=(((1,), (1,)), ((), ())),
        preferred_element_type=jnp.float32,
    )

    kf = jnp.float32(k)
    hi_g = jnp.maximum(_col_max(pt), 1e-30)
    scale = jnp.float32(_Q_LEVELS) / hi_g
    q = jnp.clip(pt * scale, 0.0, jnp.float32(_Q_LEVELS)).astype(
        jnp.int32).astype(jnp.int16)

    lo_i = jnp.zeros(hi_g.shape, jnp.int32)
    hi_i = jnp.full(hi_g.shape, 32768, jnp.int32)

    def coarse(_, carry):
        lo_i, hi_i = carry
        mid = (lo_i + hi_i) >> 1
        ge = _count_ge_i16(q, mid) >= kf
        return jnp.where(ge, mid, lo_i), jnp.where(ge, hi_i, mid)

    lo_i, _ = jax.lax.fori_loop(0, _COARSE_ITERS, coarse, (lo_i, hi_i))

    inv = hi_g * jnp.float32(1.0 / _Q_LEVELS)
    lo = lo_i.astype(jnp.float32) * inv * jnp.float32(1.0 - 1e-5)
    hi = (lo_i + 1).astype(jnp.float32) * inv * jnp.float32(1.0 + 1e-5) + 1e-30

    def fine(_, carry):
        lo, hi = carry
        mid = 0.5 * (lo + hi)
        ge = _count_ge_f32(pt, mid) >= kf
        return jnp.where(ge, mid, lo), jnp.where(ge, hi, mid)

    thr, _ = jax.lax.fori_loop(0, _FINE_ITERS, fine, (lo, hi))

    nrm = jnp.sqrt(_sq_sum_masked(pt, thr))
    inv_n = 1.0 / jnp.maximum(nrm, 1e-12)
    out_t = jnp.where(pt >= thr, pt, 0.0) * inv_n
    o_ref[...] = out_t.T


def kernel(x, projection_weights):
    n, d_in = x.shape
    d_out = projection_weights.shape[0]
    k = max(1, int(d_out * 0.1))

    min_val = jnp.min(x).reshape(1, 1)

    block_rows = 512
    grid = (n // block_rows,)

    return pl.pallas_call(
        functools.partial(_fused_kernel, k=k),
        grid=grid,
        in_specs=[
            pl.BlockSpec(memory_space=pltpu.SMEM),
            pl.BlockSpec((block_rows, d_in), lambda i: (i, 0)),
            pl.BlockSpec((d_out, d_in), lambda i: (0, 0)),
        ],
        out_specs=pl.BlockSpec((block_rows, d_out), lambda i: (i, 0)),
        out_shape=jax.ShapeDtypeStruct((n, d_out), jnp.float32),
        compiler_params=pltpu.CompilerParams(
            dimension_semantics=("arbitrary",),
        ),
    )(min_val, x, projection_weights)


# speedup vs baseline: 1.1019x; 1.0008x over previous
import functools

import jax
import jax.numpy as jnp
from jax.experimental import pallas as pl
from jax.experimental.pallas import tpu as pltpu

_Q_LEVELS = 32704
_COARSE_ITERS = 15
_FINE_ITERS = 4
_NACC = 8


def _fold8(a):
    a = a[:4] + a[4:8]
    a = a[:2] + a[2:4]
    return a[:1] + a[1:2]


def _count_ge_f32(pt, mid):
    rows, r = pt.shape
    mid8 = jnp.broadcast_to(mid, (8, r))
    accs = [jnp.zeros((8, r), jnp.float32) for _ in range(_NACC)]
    for c in range(rows // 8):
        sl = pt[8 * c:8 * c + 8]
        accs[c % _NACC] += (sl >= mid8).astype(jnp.float32)
    a = accs[0]
    for c in range(1, _NACC):
        a += accs[c]
    return _fold8(a)


def _count_ge_i16(q, mid_i):
    rows, r = q.shape
    mid16 = jnp.broadcast_to(mid_i.astype(jnp.int16), (16, r))
    one = jnp.int16(1)
    zero = jnp.int16(0)
    accs = [jnp.zeros((16, r), jnp.int16) for _ in range(_NACC)]
    for c in range(rows // 16):
        sl = q[16 * c:16 * c + 16]
        accs[c % _NACC] += jnp.where(sl >= mid16, one, zero)
    a = accs[0]
    for c in range(1, _NACC):
        a += accs[c]
    a = a[:8] + a[8:16]
    a = a[:4] + a[4:8]
    a = a[:2] + a[2:4]
    af = a.astype(jnp.float32)
    return af[:1] + af[1:2]


def _col_max(pt):
    rows, r = pt.shape
    accs = [jnp.zeros((8, r), jnp.float32) for _ in range(_NACC)]
    for c in range(rows // 8):
        sl = pt[8 * c:8 * c + 8]
        accs[c % _NACC] = jnp.maximum(accs[c % _NACC], sl)
    a = accs[0]
    for c in range(1, _NACC):
        a = jnp.maximum(a, accs[c])
    a = jnp.maximum(a[:4], a[4:8])
    a = jnp.maximum(a[:2], a[2:4])
    return jnp.maximum(a[:1], a[1:2])


def _sq_sum_masked(pt, thr):
    rows, r = pt.shape
    thr8 = jnp.broadcast_to(thr, (8, r))
    zero = jnp.float32(0.0)
    accs = [jnp.zeros((8, r), jnp.float32) for _ in range(_NACC)]
    for c in range(rows // 8):
        sl = pt[8 * c:8 * c + 8]
        e = jnp.where(sl >= thr8, sl, zero)
        accs[c % _NACC] += e * e
    a = accs[0]
    for c in range(1, _NACC):
        a += accs[c]
    return _fold8(a)


def _fused_kernel(min_ref, x_ref, w_ref, o_ref, *, k):
    xb = x_ref[...]
    xb = jnp.where(min_ref[0, 0] >= 0.0, xb * 2.0 - 1.0, xb)
    pt = jax.lax.dot_general(
        w_ref[...], xb,
        dimension_numbers=(((1,), (1,)), ((), ())),
        preferred_element_type=jnp.float32,
    )

    kf = jnp.float32(k)
    hi_g = jnp.maximum(_col_max(pt), 1e-30)
    scale = jnp.float32(_Q_LEVELS) / hi_g
    q = jnp.clip(pt * scale, 0.0, jnp.float32(_Q_LEVELS)).astype(
        jnp.int32).astype(jnp.int16)

    lo_i = jnp.zeros(hi_g.shape, jnp.int32)
    hi_i = jnp.full(hi_g.shape, 32768, jnp.int32)

    def coarse(_, carry):
        lo_i, hi_i = carry
        mid = (lo_i + hi_i) >> 1
        ge = _count_ge_i16(q, mid) >= kf
        return jnp.where(ge, mid, lo_i), jnp.where(ge, hi_i, mid)

    lo_i, _ = jax.lax.fori_loop(0, _COARSE_ITERS, coarse, (lo_i, hi_i))

    inv = hi_g * jnp.float32(1.0 / _Q_LEVELS)
    lo = lo_i.astype(jnp.float32) * inv * jnp.float32(1.0 - 1e-5)
    hi = (lo_i + 1).astype(jnp.float32) * inv * jnp.float32(1.0 + 1e-5) + 1e-30

    def fine(_, carry):
        lo, hi = carry
        mid = 0.5 * (lo + hi)
        ge = _count_ge_f32(pt, mid) >= kf
        return jnp.where(ge, mid, lo), jnp.where(ge, hi, mid)

    thr, _ = jax.lax.fori_loop(0, _FINE_ITERS, fine, (lo, hi))

    nrm = jnp.sqrt(_sq_sum_masked(pt, thr))
    inv_n = 1.0 / jnp.maximum(nrm, 1e-12)
    out_t = jnp.where(pt >= thr, pt, 0.0) * inv_n
    o_ref[...] = out_t.T


def kernel(x, projection_weights):
    n, d_in = x.shape
    d_out = projection_weights.shape[0]
    k = max(1, int(d_out * 0.1))

    min_val = jnp.min(x).reshape(1, 1)

    block_rows = 512
    grid = (n // block_rows,)

    return pl.pallas_call(
        functools.partial(_fused_kernel, k=k),
        grid=grid,
        in_specs=[
            pl.BlockSpec(memory_space=pltpu.SMEM),
            pl.BlockSpec((block_rows, d_in), lambda i: (i, 0)),
            pl.BlockSpec((d_out, d_in), lambda i: (0, 0)),
        ],
        out_specs=pl.BlockSpec((block_rows, d_out), lambda i: (i, 0)),
        out_shape=jax.ShapeDtypeStruct((n, d_out), jnp.float32),
        compiler_params=pltpu.CompilerParams(
            dimension_semantics=("parallel",),
        ),
    )(min_val, x, projection_weights)
